# initial kernel scaffold (unmeasured)
import jax
import jax.numpy as jnp
from jax import lax
from jax.experimental import pallas as pl
from jax.experimental.pallas import tpu as pltpu


def kernel(
    x,
):
    def body(*refs):
        pass

    out_shape = jax.ShapeDtypeStruct(..., jnp.float32)
    return pl.pallas_call(body, out_shape=out_shape)(...)


# baseline (device time: 54185 ns/iter reference)
import jax
import jax.numpy as jnp
from jax import lax
from jax.experimental import pallas as pl
from jax.experimental.pallas import tpu as pltpu

N_XDEV = 2


def kernel(x):
    m_per, n = x.shape

    def body(x_ref, out_ref, send_sem, recv_sem):
        my_x = lax.axis_index("x")
        my_y = lax.axis_index("y")
        nbr = (1 - my_x, my_y)

        barrier_sem = pltpu.get_barrier_semaphore()
        pl.semaphore_signal(
            barrier_sem, inc=1, device_id=nbr,
            device_id_type=pl.DeviceIdType.MESH,
        )
        pl.semaphore_wait(barrier_sem, 1)

        out_ref[pl.ds(my_x * m_per, m_per), :] = x_ref[...]

        send = pltpu.make_async_remote_copy(
            src_ref=x_ref,
            dst_ref=out_ref.at[pl.ds(my_x * m_per, m_per), :],
            send_sem=send_sem,
            recv_sem=recv_sem,
            device_id=nbr,
            device_id_type=pl.DeviceIdType.MESH,
        )
        send.start()

        recv = pltpu.make_async_remote_copy(
            src_ref=x_ref,
            dst_ref=out_ref.at[pl.ds((1 - my_x) * m_per, m_per), :],
            send_sem=send_sem,
            recv_sem=recv_sem,
            device_id=nbr,
            device_id_type=pl.DeviceIdType.MESH,
        )
        send.wait_send()
        recv.wait_recv()

    out_shape = jax.ShapeDtypeStruct((N_XDEV * m_per, n), x.dtype)
    return pl.pallas_call(
        body,
        out_shape=out_shape,
        in_specs=[pl.BlockSpec(memory_space=pltpu.VMEM)],
        out_specs=pl.BlockSpec(memory_space=pltpu.VMEM),
        scratch_shapes=[
            pltpu.SemaphoreType.DMA,
            pltpu.SemaphoreType.DMA,
        ],
        compiler_params=pltpu.CompilerParams(collective_id=0),
    )(x)


# device time: 36956 ns/iter; 1.4662x vs baseline; 1.4662x over previous
import jax
import jax.numpy as jnp
from jax import lax
from jax.experimental import pallas as pl
from jax.experimental.pallas import tpu as pltpu

N_XDEV = 2
NB = 8


def kernel(x):
    m_per, n = x.shape
    half = m_per // 2
    blk = half // NB

    def body(x_ref, out_ref, x_send_sems, x_recv_sems, y_send_sems, y_recv_sems):
        my_x = lax.axis_index("x")
        my_y = lax.axis_index("y")
        x_nbr = (1 - my_x, my_y)
        y_nbr = (my_x, 1 - my_y)

        my_half_rows = my_x * m_per + my_y * half
        x_recv_rows = (1 - my_x) * m_per + my_y * half
        y_recv_rows = (1 - my_x) * m_per + (1 - my_y) * half

        barrier_sem = pltpu.get_barrier_semaphore()
        for nbr in (x_nbr, y_nbr):
            pl.semaphore_signal(
                barrier_sem, inc=1, device_id=nbr,
                device_id_type=pl.DeviceIdType.MESH,
            )
        pl.semaphore_wait(barrier_sem, 2)

        x_sends = []
        for b in range(NB):
            s = pltpu.make_async_remote_copy(
                src_ref=x_ref.at[pl.ds(my_y * half + b * blk, blk), :],
                dst_ref=out_ref.at[pl.ds(my_half_rows + b * blk, blk), :],
                send_sem=x_send_sems.at[b],
                recv_sem=x_recv_sems.at[b],
                device_id=x_nbr,
                device_id_type=pl.DeviceIdType.MESH,
            )
            s.start()
            x_sends.append(s)

        out_ref[pl.ds(my_x * m_per, m_per), :] = x_ref[...]

        y_sends = []
        for b in range(NB):
            recv = pltpu.make_async_remote_copy(
                src_ref=x_ref.at[pl.ds(b * blk, blk), :],
                dst_ref=out_ref.at[pl.ds(x_recv_rows + b * blk, blk), :],
                send_sem=x_send_sems.at[b],
                recv_sem=x_recv_sems.at[b],
                device_id=x_nbr,
                device_id_type=pl.DeviceIdType.MESH,
            )
            recv.wait_recv()
            fwd = pltpu.make_async_remote_copy(
                src_ref=out_ref.at[pl.ds(x_recv_rows + b * blk, blk), :],
                dst_ref=out_ref.at[pl.ds(x_recv_rows + b * blk, blk), :],
                send_sem=y_send_sems.at[b],
                recv_sem=y_recv_sems.at[b],
                device_id=y_nbr,
                device_id_type=pl.DeviceIdType.MESH,
            )
            fwd.start()
            y_sends.append(fwd)

        for b in range(NB):
            recv = pltpu.make_async_remote_copy(
                src_ref=x_ref.at[pl.ds(b * blk, blk), :],
                dst_ref=out_ref.at[pl.ds(y_recv_rows + b * blk, blk), :],
                send_sem=y_send_sems.at[b],
                recv_sem=y_recv_sems.at[b],
                device_id=y_nbr,
                device_id_type=pl.DeviceIdType.MESH,
            )
            recv.wait_recv()

        for s in x_sends:
            s.wait_send()
        for s in y_sends:
            s.wait_send()

    out_shape = jax.ShapeDtypeStruct((N_XDEV * m_per, n), x.dtype)
    return pl.pallas_call(
        body,
        out_shape=out_shape,
        in_specs=[pl.BlockSpec(memory_space=pltpu.VMEM)],
        out_specs=pl.BlockSpec(memory_space=pltpu.VMEM),
        scratch_shapes=[
            pltpu.SemaphoreType.DMA((NB,)),
            pltpu.SemaphoreType.DMA((NB,)),
            pltpu.SemaphoreType.DMA((NB,)),
            pltpu.SemaphoreType.DMA((NB,)),
        ],
        compiler_params=pltpu.CompilerParams(collective_id=0),
    )(x)


# device time: 36182 ns/iter; 1.4976x vs baseline; 1.0214x over previous
import jax
import jax.numpy as jnp
from jax import lax
from jax.experimental import pallas as pl
from jax.experimental.pallas import tpu as pltpu

N_XDEV = 2
NB = 16


def kernel(x):
    m_per, n = x.shape
    half = m_per // 2
    blk = half // NB

    def body(x_ref, out_ref, x_send_sems, x_recv_sems, y_send_sems,
             y_recv_sems, local_sem):
        my_x = lax.axis_index("x")
        my_y = lax.axis_index("y")
        x_nbr = (1 - my_x, my_y)
        y_nbr = (my_x, 1 - my_y)

        my_half_rows = my_x * m_per + my_y * half
        x_recv_rows = (1 - my_x) * m_per + my_y * half
        y_recv_rows = (1 - my_x) * m_per + (1 - my_y) * half

        local = pltpu.make_async_copy(
            x_ref, out_ref.at[pl.ds(my_x * m_per, m_per), :], local_sem
        )
        local.start()

        barrier_sem = pltpu.get_barrier_semaphore()
        for nbr in (x_nbr, y_nbr):
            pl.semaphore_signal(
                barrier_sem, inc=1, device_id=nbr,
                device_id_type=pl.DeviceIdType.MESH,
            )
        pl.semaphore_wait(barrier_sem, 2)

        x_sends = []
        for b in range(NB):
            s = pltpu.make_async_remote_copy(
                src_ref=x_ref.at[pl.ds(my_y * half + b * blk, blk), :],
                dst_ref=out_ref.at[pl.ds(my_half_rows + b * blk, blk), :],
                send_sem=x_send_sems.at[b],
                recv_sem=x_recv_sems.at[b],
                device_id=x_nbr,
                device_id_type=pl.DeviceIdType.MESH,
            )
            s.start()
            x_sends.append(s)

        y_sends = []
        for b in range(NB):
            recv = pltpu.make_async_remote_copy(
                src_ref=x_ref.at[pl.ds(b * blk, blk), :],
                dst_ref=out_ref.at[pl.ds(x_recv_rows + b * blk, blk), :],
                send_sem=x_send_sems.at[b],
                recv_sem=x_recv_sems.at[b],
                device_id=x_nbr,
                device_id_type=pl.DeviceIdType.MESH,
            )
            recv.wait_recv()
            fwd = pltpu.make_async_remote_copy(
                src_ref=out_ref.at[pl.ds(x_recv_rows + b * blk, blk), :],
                dst_ref=out_ref.at[pl.ds(x_recv_rows + b * blk, blk), :],
                send_sem=y_send_sems.at[b],
                recv_sem=y_recv_sems.at[b],
                device_id=y_nbr,
                device_id_type=pl.DeviceIdType.MESH,
            )
            fwd.start()
            y_sends.append(fwd)

        for b in range(NB):
            recv = pltpu.make_async_remote_copy(
                src_ref=x_ref.at[pl.ds(b * blk, blk), :],
                dst_ref=out_ref.at[pl.ds(y_recv_rows + b * blk, blk), :],
                send_sem=y_send_sems.at[b],
                recv_sem=y_recv_sems.at[b],
                device_id=y_nbr,
                device_id_type=pl.DeviceIdType.MESH,
            )
            recv.wait_recv()

        local.wait()
        for s in x_sends:
            s.wait_send()
        for s in y_sends:
            s.wait_send()

    out_shape = jax.ShapeDtypeStruct((N_XDEV * m_per, n), x.dtype)
    return pl.pallas_call(
        body,
        out_shape=out_shape,
        in_specs=[pl.BlockSpec(memory_space=pl.ANY)],
        out_specs=pl.BlockSpec(memory_space=pl.ANY),
        scratch_shapes=[
            pltpu.SemaphoreType.DMA((NB,)),
            pltpu.SemaphoreType.DMA((NB,)),
            pltpu.SemaphoreType.DMA((NB,)),
            pltpu.SemaphoreType.DMA((NB,)),
            pltpu.SemaphoreType.DMA,
        ],
        compiler_params=pltpu.CompilerParams(collective_id=0),
    )(x)


# device time: 36153 ns/iter; 1.4988x vs baseline; 1.0008x over previous
import jax
import jax.numpy as jnp
from jax import lax
from jax.experimental import pallas as pl
from jax.experimental.pallas import tpu as pltpu

N_XDEV = 2
NB = 16


def kernel(x):
    m_per, n = x.shape
    half = m_per // 2
    blk = half // NB

    def body(x_ref, out_ref, x_send_sems, x_recv_sems, y_send_sems,
             y_recv_sems, local_sem):
        my_x = lax.axis_index("x")
        my_y = lax.axis_index("y")
        x_nbr = (1 - my_x, my_y)
        y_nbr = (my_x, 1 - my_y)

        my_half_rows = my_x * m_per + my_y * half
        x_recv_rows = (1 - my_x) * m_per + my_y * half
        y_recv_rows = (1 - my_x) * m_per + (1 - my_y) * half

        local = pltpu.make_async_copy(
            x_ref, out_ref.at[pl.ds(my_x * m_per, m_per), :], local_sem
        )
        local.start()

        barrier_sem = pltpu.get_barrier_semaphore()
        for nbr in (x_nbr, y_nbr):
            pl.semaphore_signal(
                barrier_sem, inc=1, device_id=nbr,
                device_id_type=pl.DeviceIdType.MESH,
            )
        pl.semaphore_wait(barrier_sem, 2)

        x_sends = []
        for b in range(NB):
            s = pltpu.make_async_remote_copy(
                src_ref=x_ref.at[pl.ds(my_y * half + b * blk, blk), :],
                dst_ref=out_ref.at[pl.ds(my_half_rows + b * blk, blk), :],
                send_sem=x_send_sems.at[b],
                recv_sem=x_recv_sems.at[b],
                device_id=x_nbr,
                device_id_type=pl.DeviceIdType.MESH,
            )
            s.start()
            x_sends.append(s)

        y_sends = []
        for b in range(NB):
            recv = pltpu.make_async_remote_copy(
                src_ref=x_ref.at[pl.ds(b * blk, blk), :],
                dst_ref=out_ref.at[pl.ds(x_recv_rows + b * blk, blk), :],
                send_sem=x_send_sems.at[b],
                recv_sem=x_recv_sems.at[b],
                device_id=x_nbr,
                device_id_type=pl.DeviceIdType.MESH,
            )
            recv.wait_recv()
            fwd = pltpu.make_async_remote_copy(
                src_ref=out_ref.at[pl.ds(x_recv_rows + b * blk, blk), :],
                dst_ref=out_ref.at[pl.ds(x_recv_rows + b * blk, blk), :],
                send_sem=y_send_sems.at[b],
                recv_sem=y_recv_sems.at[b],
                device_id=y_nbr,
                device_id_type=pl.DeviceIdType.MESH,
            )
            fwd.start()
            y_sends.append(fwd)

        for b in range(NB):
            recv = pltpu.make_async_remote_copy(
                src_ref=x_ref.at[pl.ds(b * blk, blk), :],
                dst_ref=out_ref.at[pl.ds(y_recv_rows + b * blk, blk), :],
                send_sem=y_send_sems.at[b],
                recv_sem=y_recv_sems.at[b],
                device_id=y_nbr,
                device_id_type=pl.DeviceIdType.MESH,
            )
            recv.wait_recv()

        local.wait()
        for s in x_sends:
            s.wait_send()
        for s in y_sends:
            s.wait_send()

    out_shape = jax.ShapeDtypeStruct((N_XDEV * m_per, n), x.dtype)
    return pl.pallas_call(
        body,
        out_shape=out_shape,
        in_specs=[pl.BlockSpec(memory_space=pltpu.MemorySpace.HBM)],
        out_specs=pl.BlockSpec(memory_space=pltpu.MemorySpace.HBM),
        scratch_shapes=[
            pltpu.SemaphoreType.DMA((NB,)),
            pltpu.SemaphoreType.DMA((NB,)),
            pltpu.SemaphoreType.DMA((NB,)),
            pltpu.SemaphoreType.DMA((NB,)),
            pltpu.SemaphoreType.DMA,
        ],
        compiler_params=pltpu.CompilerParams(collective_id=0),
    )(x)
